# baseline (speedup 1.0000x reference)
import functools

import jax
import jax.numpy as jnp
from jax import lax
from jax.experimental import pallas as pl
from jax.experimental.pallas import tpu as pltpu
from jax.experimental.pallas import tpu_sc as plsc

VOCAB = 100000
D = 1024
BATCH = 4
SEQ = 2048
NC, NS = 2, 16
NW = NC * NS
PP = SEQ // NW
SP = 8
NSTAGE = PP // SP
NSET = 3
ROWS = SP * BATCH
LANES = 16

_mesh = plsc.VectorSubcoreMesh(core_axis_name="c", subcore_axis_name="s")


@functools.partial(
    pl.kernel,
    mesh=_mesh,
    out_type=jax.ShapeDtypeStruct((BATCH, SEQ, D), jnp.float32),
    scratch_types=(
        [pltpu.VMEM((NSTAGE * ROWS,), jnp.int32)]
        + [pltpu.VMEM((ROWS, D), jnp.float32) for _ in range(NSET)]
        + [pltpu.VMEM((SP, D), jnp.float32) for _ in range(2)]
        + [pltpu.SemaphoreType.DMA for _ in range(NSET + NSET + 2 + 1)]
    ),
)
def _embed(x_hbm, tok_hbm, pos_hbm, out_hbm, idx_v, *rest):
    bufs = rest[:NSET]
    poss = rest[NSET:NSET + 2]
    gsems = rest[NSET + 2:2 * NSET + 2]
    wsems = rest[2 * NSET + 2:3 * NSET + 2]
    psems = rest[3 * NSET + 2:3 * NSET + 4]
    isem = rest[3 * NSET + 4]

    wid = lax.axis_index("s") * NC + lax.axis_index("c")
    p_base = wid * PP

    for t in range(NSTAGE):
        for b in range(BATCH):
            pltpu.async_copy(
                x_hbm.at[b, pl.ds(p_base + t * SP, SP)],
                idx_v.at[pl.ds(t * ROWS + b * SP, SP)],
                isem,
            )
    pltpu.make_async_copy(x_hbm.at[0, pl.ds(0, NSTAGE * ROWS)], idx_v, isem).wait()

    def load_pos(t):
        return pltpu.async_copy(
            pos_hbm.at[pl.ds(p_base + t * SP, SP)], poss[t % 2], psems[t % 2]
        )

    def gather_stage(t):
        s = t % NSET
        return pltpu.async_copy(
            tok_hbm.at[idx_v.at[pl.ds(t * ROWS, ROWS)]], bufs[s], gsems[s]
        )

    h_pos = [None] * NSTAGE
    for t in range(2):
        h_pos[t] = load_pos(t)
    h_g = [None] * NSTAGE
    for t in range(NSET):
        h_g[t] = gather_stage(t)

    def drain_writes(s):
        pltpu.make_async_copy(
            tok_hbm.at[idx_v.at[pl.ds(0, ROWS)]], bufs[s], wsems[s]
        ).wait()

    for t in range(NSTAGE):
        s = t % NSET
        h_g[t].wait()
        h_pos[t].wait()
        buf = bufs[s]
        pbuf = poss[t % 2]

        def _row(i, carry):
            for k in range(D // LANES):
                sl = pl.ds(k * LANES, LANES)
                v = pbuf[i, sl]
                for b in range(BATCH):
                    plsc.addupdate(buf.at[b * SP + i, sl], v)
            return carry

        lax.fori_loop(0, SP, _row, 0)
        for b in range(BATCH):
            pltpu.async_copy(
                buf.at[pl.ds(b * SP, SP)],
                out_hbm.at[b, pl.ds(p_base + t * SP, SP)],
                wsems[s],
            )
        if t + 2 < NSTAGE:
            h_pos[t + 2] = load_pos(t + 2)
        if NSET <= t + 2 < NSTAGE:
            drain_writes((t + 2) % NSET)
            h_g[t + 2] = gather_stage(t + 2)

    for t in range(NSTAGE - NSET, NSTAGE):
        drain_writes(t % NSET)


def kernel(x, token_table, position_table):
    return _embed(x.astype(jnp.int32), token_table, position_table)
